# baseline (speedup 1.0000x reference)
import functools

import jax
import jax.numpy as jnp
from jax import lax
from jax.experimental import pallas as pl
from jax.experimental.pallas import tpu as pltpu
from jax.experimental.pallas import tpu_sc as plsc

S, D, F, E, CAP = 2048, 1024, 2048, 16, 160
SLOTS = E * CAP
TB = 256
FB = 1024
NF = F // FB

_INTERPRET = False


def _enc_body(x_ref, w_ref, b_ref, o_ref):
    o_ref[...] = (
        jnp.dot(x_ref[...], w_ref[...], preferred_element_type=jnp.float32)
        + b_ref[...]
    )


def _encoder(xf, W_enc, b_enc):
    return pl.pallas_call(
        _enc_body,
        grid=(S // TB,),
        in_specs=[
            pl.BlockSpec((TB, D), lambda i: (i, 0)),
            pl.BlockSpec((D, D), lambda i: (0, 0)),
            pl.BlockSpec((1, D), lambda i: (0, 0)),
        ],
        out_specs=pl.BlockSpec((TB, D), lambda i: (i, 0)),
        out_shape=jax.ShapeDtypeStruct((S, D), jnp.float32),
        interpret=_INTERPRET,
    )(xf, W_enc, b_enc)


def _route_body(xf_ref, wr_ref, maskf_ref, sidx_ref, gidx_ref, gain_ref,
                carry_ref):
    i = pl.program_id(0)

    @pl.when(i == 0)
    def _():
        carry_ref[...] = jnp.zeros_like(carry_ref)

    logits = jnp.dot(xf_ref[...], wr_ref[0],
                     preferred_element_type=jnp.float32)
    m = jnp.max(logits, axis=1, keepdims=True)
    p = jnp.exp(logits - m)
    ssum = jnp.sum(p, axis=1, keepdims=True)
    maskf = maskf_ref[...]
    gate = maskf / ssum

    lane = lax.broadcasted_iota(jnp.int32, (TB, E), 1)
    eidx = jnp.min(jnp.where(logits == m, lane, E), axis=1, keepdims=True)
    onehot = ((lane == eidx) & (maskf > 0)).astype(jnp.float32)

    row = lax.broadcasted_iota(jnp.int32, (TB, TB), 0)
    col = lax.broadcasted_iota(jnp.int32, (TB, TB), 1)
    tri = (col < row).astype(jnp.float32)
    local = jnp.dot(tri, onehot, preferred_element_type=jnp.float32)
    posfull = local + carry_ref[...]
    pos = jnp.sum(posfull * onehot, axis=1, keepdims=True)
    carry_ref[...] = carry_ref[...] + jnp.sum(onehot, axis=0, keepdims=True)

    keep = (pos < CAP) & (maskf > 0)
    keepf = keep.astype(jnp.float32)
    posc = jnp.minimum(pos, CAP - 1).astype(jnp.int32)
    slot = eidx * CAP + posc
    sidx_ref[...] = jnp.where(keep, slot, SLOTS)
    gidx_ref[...] = jnp.where(keep, slot, 0)
    gain_ref[...] = jnp.broadcast_to(gate * keepf, (TB, E))


def _route(xf, Wr, maskf, l):
    return pl.pallas_call(
        _route_body,
        grid=(S // TB,),
        in_specs=[
            pl.BlockSpec((TB, D), lambda i: (i, 0)),
            pl.BlockSpec((1, D, E), lambda i: (l, 0, 0)),
            pl.BlockSpec((TB, 1), lambda i: (i, 0)),
        ],
        out_specs=[
            pl.BlockSpec((TB, 1), lambda i: (i, 0)),
            pl.BlockSpec((TB, 1), lambda i: (i, 0)),
            pl.BlockSpec((TB, E), lambda i: (i, 0)),
        ],
        out_shape=[
            jax.ShapeDtypeStruct((S, 1), jnp.int32),
            jax.ShapeDtypeStruct((S, 1), jnp.int32),
            jax.ShapeDtypeStruct((S, E), jnp.float32),
        ],
        scratch_shapes=[pltpu.VMEM((1, E), jnp.float32)],
        interpret=_INTERPRET,
    )(xf, Wr, maskf)


NSPLIT = 8
DQ = D // NSPLIT
FQ = F // NSPLIT


NC, NS = 2, 16
NW = NC * NS
TPW = S // NW
HTPW = TPW // 2
BUFROWS = SLOTS + 8

_sc_mesh = plsc.VectorSubcoreMesh(core_axis_name="c", subcore_axis_name="s")


@functools.partial(
    pl.kernel, mesh=_sc_mesh,
    out_type=jax.ShapeDtypeStruct((BUFROWS, D), jnp.float32),
    scratch_types=[
        pltpu.VMEM((TPW,), jnp.int32),
        pltpu.VMEM((TPW, D), jnp.float32),
        pltpu.SemaphoreType.DMA,
    ],
)
def _sc_scatter(xf_hbm, sidx_hbm, buf_hbm, idx_v, rows_v, sem):
    wid = lax.axis_index("s") * NC + lax.axis_index("c")
    base = wid * TPW
    pltpu.sync_copy(sidx_hbm.at[pl.ds(base, TPW)], idx_v)
    pltpu.sync_copy(xf_hbm.at[pl.ds(base, TPW)], rows_v)
    pltpu.async_copy(rows_v, buf_hbm.at[idx_v], sem).wait()


@functools.partial(
    pl.kernel, mesh=_sc_mesh,
    out_type=jax.ShapeDtypeStruct((S, D), jnp.float32),
    scratch_types=[
        pltpu.VMEM((HTPW,), jnp.int32),
        pltpu.VMEM((HTPW, E), jnp.float32),
        pltpu.VMEM((HTPW, D), jnp.float32),
        pltpu.VMEM((HTPW, D), jnp.float32),
        pltpu.SemaphoreType.DMA,
    ],
)
def _sc_combine(xf_hbm, ob_hbm, gidx_hbm, gain_hbm, out_hbm,
                idx_v, gain_v, xrows_v, obrows_v, sem):
    wid = lax.axis_index("s") * NC + lax.axis_index("c")
    for half in range(2):
        base = wid * TPW + half * HTPW
        pltpu.sync_copy(gidx_hbm.at[pl.ds(base, HTPW)], idx_v)
        pltpu.sync_copy(gain_hbm.at[pl.ds(base, HTPW)], gain_v)
        pltpu.sync_copy(xf_hbm.at[pl.ds(base, HTPW)], xrows_v)
        pltpu.async_copy(ob_hbm.at[idx_v], obrows_v, sem).wait()

        def row_body(r, _):
            g = gain_v[r, pl.ds(0, 16)]
            for c in range(D // 16):
                sl = pl.ds(c * 16, 16)
                xrows_v[r, sl] = xrows_v[r, sl] + g * obrows_v[r, sl]
            return 0

        lax.fori_loop(0, HTPW, row_body, 0)
        pltpu.sync_copy(xrows_v, out_hbm.at[pl.ds(base, HTPW)])


def _ffn_body(buf_ref, *rest):
    w1_refs = rest[0:NSPLIT]
    b1_ref = rest[NSPLIT]
    w2_refs = rest[NSPLIT + 1:2 * NSPLIT + 1]
    b2_ref = rest[2 * NSPLIT + 1]
    o_ref = rest[2 * NSPLIT + 2]
    buf = buf_ref[...].astype(jnp.bfloat16)
    h = b1_ref[0, 0].astype(jnp.float32)
    for q in range(NSPLIT):
        h = h + jnp.dot(buf[:, q * DQ:(q + 1) * DQ],
                        w1_refs[q][0, 0].astype(jnp.bfloat16),
                        preferred_element_type=jnp.float32)
    h = jnp.maximum(h, 0.0).astype(jnp.bfloat16)
    o = b2_ref[0, 0].astype(jnp.float32)
    for q in range(NSPLIT):
        o = o + jnp.dot(h[:, q * FQ:(q + 1) * FQ],
                        w2_refs[q][0, 0].astype(jnp.bfloat16),
                        preferred_element_type=jnp.float32)
    o_ref[...] = o


def _ffn(buf, W1, b1, W2, b2, l):
    w1_specs = [
        pl.BlockSpec((1, 1, DQ, F), (lambda e, q=q: (l, e, q, 0)))
        for q in range(NSPLIT)
    ]
    w2_specs = [
        pl.BlockSpec((1, 1, FQ, D), (lambda e, q=q: (l, e, q, 0)))
        for q in range(NSPLIT)
    ]
    return pl.pallas_call(
        _ffn_body,
        grid=(E,),
        in_specs=[
            pl.BlockSpec((CAP, D), lambda e: (e, 0)),
            *w1_specs,
            pl.BlockSpec((1, 1, 1, F), lambda e: (l, e, 0, 0)),
            *w2_specs,
            pl.BlockSpec((1, 1, 1, D), lambda e: (l, e, 0, 0)),
        ],
        out_specs=pl.BlockSpec((CAP, D), lambda e: (e, 0)),
        out_shape=jax.ShapeDtypeStruct((SLOTS, D), jnp.float32),
        interpret=_INTERPRET,
    )(buf,
      *([W1] * NSPLIT), b1,
      *([W2] * NSPLIT), b2)


def kernel(x, attention_mask, W_enc, b_enc, Wr, W1, b1, W2, b2):
    xf = _encoder(x.reshape(S, D), W_enc, b_enc.reshape(1, D))
    maskf = attention_mask.reshape(S, 1).astype(jnp.float32)
    L = Wr.shape[0]
    b1r = b1.reshape(L, E, 1, F)
    b2r = b2.reshape(L, E, 1, D)
    for l in range(L):
        sidx, gidx, gain = _route(xf, Wr, maskf, l)
        buf = _sc_scatter(xf, sidx.reshape(S))
        ob = _ffn(buf, W1, b1r, W2, b2r, l)
        xf = _sc_combine(xf, ob, gidx.reshape(S), gain)
    return xf.reshape(1, S, D)


# speedup vs baseline: 1.0256x; 1.0256x over previous
import functools

import jax
import jax.numpy as jnp
from jax import lax
from jax.experimental import pallas as pl
from jax.experimental.pallas import tpu as pltpu
from jax.experimental.pallas import tpu_sc as plsc

S, D, F, E, CAP = 2048, 1024, 2048, 16, 160
SLOTS = E * CAP
TB = 256
FB = 1024
NF = F // FB

_INTERPRET = False


def _enc_body(x_ref, w_ref, b_ref, o_ref):
    o_ref[...] = (
        jnp.dot(x_ref[...], w_ref[...], preferred_element_type=jnp.float32)
        + b_ref[...]
    )


def _encoder(xf, W_enc, b_enc):
    return pl.pallas_call(
        _enc_body,
        grid=(S // TB,),
        in_specs=[
            pl.BlockSpec((TB, D), lambda i: (i, 0)),
            pl.BlockSpec((D, D), lambda i: (0, 0)),
            pl.BlockSpec((1, D), lambda i: (0, 0)),
        ],
        out_specs=pl.BlockSpec((TB, D), lambda i: (i, 0)),
        out_shape=jax.ShapeDtypeStruct((S, D), jnp.float32),
        interpret=_INTERPRET,
    )(xf, W_enc, b_enc)


def _encroute_body(x_ref, w_ref, b_ref, wr_ref, maskf_ref,
                   hid_ref, sidx_ref, gidx_ref, gain_ref, carry_ref):
    i = pl.program_id(0)

    @pl.when(i == 0)
    def _():
        carry_ref[...] = jnp.zeros_like(carry_ref)

    hid = (jnp.dot(x_ref[...], w_ref[...], preferred_element_type=jnp.float32)
           + b_ref[...])
    hid_ref[...] = hid
    logits = jnp.dot(hid, wr_ref[0], preferred_element_type=jnp.float32)
    _route_core(logits, maskf_ref, sidx_ref, gidx_ref, gain_ref, carry_ref)


def _encroute(xf, W_enc, b_enc, Wr, maskf):
    return pl.pallas_call(
        _encroute_body,
        grid=(S // TB,),
        in_specs=[
            pl.BlockSpec((TB, D), lambda i: (i, 0)),
            pl.BlockSpec((D, D), lambda i: (0, 0)),
            pl.BlockSpec((1, D), lambda i: (0, 0)),
            pl.BlockSpec((1, D, E), lambda i: (0, 0, 0)),
            pl.BlockSpec((TB, 1), lambda i: (i, 0)),
        ],
        out_specs=[
            pl.BlockSpec((TB, D), lambda i: (i, 0)),
            pl.BlockSpec((TB, 1), lambda i: (i, 0)),
            pl.BlockSpec((TB, 1), lambda i: (i, 0)),
            pl.BlockSpec((TB, E), lambda i: (i, 0)),
        ],
        out_shape=[
            jax.ShapeDtypeStruct((S, D), jnp.float32),
            jax.ShapeDtypeStruct((S, 1), jnp.int32),
            jax.ShapeDtypeStruct((S, 1), jnp.int32),
            jax.ShapeDtypeStruct((S, E), jnp.float32),
        ],
        scratch_shapes=[pltpu.VMEM((1, E), jnp.float32)],
        interpret=_INTERPRET,
    )(xf, W_enc, b_enc, Wr, maskf)


def _route_core(logits, maskf_ref, sidx_ref, gidx_ref, gain_ref, carry_ref):
    m = jnp.max(logits, axis=1, keepdims=True)
    p = jnp.exp(logits - m)
    ssum = jnp.sum(p, axis=1, keepdims=True)
    maskf = maskf_ref[...]
    gate = maskf / ssum

    lane = lax.broadcasted_iota(jnp.int32, (TB, E), 1)
    eidx = jnp.min(jnp.where(logits == m, lane, E), axis=1, keepdims=True)
    onehot = ((lane == eidx) & (maskf > 0)).astype(jnp.float32)

    row = lax.broadcasted_iota(jnp.int32, (TB, TB), 0)
    col = lax.broadcasted_iota(jnp.int32, (TB, TB), 1)
    tri = (col < row).astype(jnp.float32)
    local = jnp.dot(tri, onehot, preferred_element_type=jnp.float32)
    posfull = local + carry_ref[...]
    pos = jnp.sum(posfull * onehot, axis=1, keepdims=True)
    carry_ref[...] = carry_ref[...] + jnp.sum(onehot, axis=0, keepdims=True)

    keep = (pos < CAP) & (maskf > 0)
    keepf = keep.astype(jnp.float32)
    posc = jnp.minimum(pos, CAP - 1).astype(jnp.int32)
    slot = eidx * CAP + posc
    sidx_ref[...] = jnp.where(keep, slot, SLOTS)
    gidx_ref[...] = jnp.where(keep, slot, 0)
    gain_ref[...] = jnp.broadcast_to(gate * keepf, (TB, E))


def _route_body(xf_ref, wr_ref, maskf_ref, sidx_ref, gidx_ref, gain_ref,
                carry_ref):
    i = pl.program_id(0)

    @pl.when(i == 0)
    def _():
        carry_ref[...] = jnp.zeros_like(carry_ref)

    logits = jnp.dot(xf_ref[...], wr_ref[0],
                     preferred_element_type=jnp.float32)
    _route_core(logits, maskf_ref, sidx_ref, gidx_ref, gain_ref, carry_ref)


def _route(xf, Wr, maskf, l):
    return pl.pallas_call(
        _route_body,
        grid=(S // TB,),
        in_specs=[
            pl.BlockSpec((TB, D), lambda i: (i, 0)),
            pl.BlockSpec((1, D, E), lambda i: (l, 0, 0)),
            pl.BlockSpec((TB, 1), lambda i: (i, 0)),
        ],
        out_specs=[
            pl.BlockSpec((TB, 1), lambda i: (i, 0)),
            pl.BlockSpec((TB, 1), lambda i: (i, 0)),
            pl.BlockSpec((TB, E), lambda i: (i, 0)),
        ],
        out_shape=[
            jax.ShapeDtypeStruct((S, 1), jnp.int32),
            jax.ShapeDtypeStruct((S, 1), jnp.int32),
            jax.ShapeDtypeStruct((S, E), jnp.float32),
        ],
        scratch_shapes=[pltpu.VMEM((1, E), jnp.float32)],
        interpret=_INTERPRET,
    )(xf, Wr, maskf)


NSPLIT = 4
DQ = D // NSPLIT
FQ = F // NSPLIT


NC, NS = 2, 16
NW = NC * NS
TPW = S // NW
HTPW = TPW // 2
BUFROWS = SLOTS + 8

_sc_mesh = plsc.VectorSubcoreMesh(core_axis_name="c", subcore_axis_name="s")


@functools.partial(
    pl.kernel, mesh=_sc_mesh,
    out_type=jax.ShapeDtypeStruct((BUFROWS, D), jnp.float32),
    scratch_types=[
        pltpu.VMEM((TPW,), jnp.int32),
        pltpu.VMEM((TPW, D), jnp.float32),
        pltpu.SemaphoreType.DMA,
    ],
)
def _sc_scatter(xf_hbm, sidx_hbm, buf_hbm, idx_v, rows_v, sem):
    wid = lax.axis_index("s") * NC + lax.axis_index("c")
    base = wid * TPW
    pltpu.sync_copy(sidx_hbm.at[pl.ds(base, TPW)], idx_v)
    pltpu.sync_copy(xf_hbm.at[pl.ds(base, TPW)], rows_v)
    pltpu.async_copy(rows_v, buf_hbm.at[idx_v], sem).wait()


@functools.partial(
    pl.kernel, mesh=_sc_mesh,
    out_type=jax.ShapeDtypeStruct((S, D), jnp.float32),
    scratch_types=[
        pltpu.VMEM((HTPW,), jnp.int32),
        pltpu.VMEM((HTPW, E), jnp.float32),
        pltpu.VMEM((HTPW, D), jnp.float32),
        pltpu.VMEM((HTPW, D), jnp.float32),
        pltpu.SemaphoreType.DMA,
    ],
)
def _sc_combine(xf_hbm, ob_hbm, gidx_hbm, gain_hbm, out_hbm,
                idx_v, gain_v, xrows_v, obrows_v, sem):
    wid = lax.axis_index("s") * NC + lax.axis_index("c")
    for half in range(2):
        base = wid * TPW + half * HTPW
        pltpu.sync_copy(gidx_hbm.at[pl.ds(base, HTPW)], idx_v)
        pltpu.sync_copy(gain_hbm.at[pl.ds(base, HTPW)], gain_v)
        pltpu.sync_copy(xf_hbm.at[pl.ds(base, HTPW)], xrows_v)
        pltpu.async_copy(ob_hbm.at[idx_v], obrows_v, sem).wait()

        def row_body(r, _):
            g = gain_v[r, pl.ds(0, 16)]
            for c in range(D // 16):
                sl = pl.ds(c * 16, 16)
                xrows_v[r, sl] = xrows_v[r, sl] + g * obrows_v[r, sl]
            return 0

        lax.fori_loop(0, HTPW, row_body, 0)
        pltpu.sync_copy(xrows_v, out_hbm.at[pl.ds(base, HTPW)])


def _ffn_body(buf_ref, *rest):
    w1_refs = rest[0:NSPLIT]
    b1_ref = rest[NSPLIT]
    w2_refs = rest[NSPLIT + 1:2 * NSPLIT + 1]
    b2_ref = rest[2 * NSPLIT + 1]
    o_ref = rest[2 * NSPLIT + 2]
    buf = buf_ref[...].astype(jnp.bfloat16)
    h = b1_ref[0, 0].astype(jnp.float32)
    for q in range(NSPLIT):
        h = h + jnp.dot(buf[:, q * DQ:(q + 1) * DQ],
                        w1_refs[q][0, 0].astype(jnp.bfloat16),
                        preferred_element_type=jnp.float32)
    h = jnp.maximum(h, 0.0).astype(jnp.bfloat16)
    o = b2_ref[0, 0].astype(jnp.float32)
    for q in range(NSPLIT):
        o = o + jnp.dot(h[:, q * FQ:(q + 1) * FQ],
                        w2_refs[q][0, 0].astype(jnp.bfloat16),
                        preferred_element_type=jnp.float32)
    o_ref[...] = o


def _ffn(buf, W1, b1, W2, b2, l):
    w1_specs = [
        pl.BlockSpec((1, 1, DQ, F), (lambda e, q=q: (l, e, q, 0)))
        for q in range(NSPLIT)
    ]
    w2_specs = [
        pl.BlockSpec((1, 1, FQ, D), (lambda e, q=q: (l, e, q, 0)))
        for q in range(NSPLIT)
    ]
    return pl.pallas_call(
        _ffn_body,
        grid=(E,),
        in_specs=[
            pl.BlockSpec((CAP, D), lambda e: (e, 0)),
            *w1_specs,
            pl.BlockSpec((1, 1, 1, F), lambda e: (l, e, 0, 0)),
            *w2_specs,
            pl.BlockSpec((1, 1, 1, D), lambda e: (l, e, 0, 0)),
        ],
        out_specs=pl.BlockSpec((CAP, D), lambda e: (e, 0)),
        out_shape=jax.ShapeDtypeStruct((SLOTS, D), jnp.float32),
        interpret=_INTERPRET,
    )(buf,
      *([W1] * NSPLIT), b1,
      *([W2] * NSPLIT), b2)


def kernel(x, attention_mask, W_enc, b_enc, Wr, W1, b1, W2, b2):
    maskf = attention_mask.reshape(S, 1).astype(jnp.float32)
    L = Wr.shape[0]
    b1r = b1.reshape(L, E, 1, F)
    b2r = b2.reshape(L, E, 1, D)
    xf, sidx, gidx, gain = _encroute(
        x.reshape(S, D), W_enc, b_enc.reshape(1, D), Wr, maskf)
    for l in range(L):
        if l > 0:
            sidx, gidx, gain = _route(xf, Wr, maskf, l)
        buf = _sc_scatter(xf, sidx.reshape(S))
        ob = _ffn(buf, W1, b1r, W2, b2r, l)
        xf = _sc_combine(xf, ob, gidx.reshape(S), gain)
    return xf.reshape(1, S, D)
